# baseline (speedup 1.0000x reference)
import functools

import jax
import jax.numpy as jnp
from jax import lax
from jax.experimental import pallas as pl
from jax.experimental.pallas import tpu as pltpu
from jax.experimental.pallas import tpu_sc as plsc

_T, _D, _H, _E, _K = 2048, 2048, 1024, 8, 2
_N = _T * _K
_B = 512
_BSH = _B.bit_length() - 1
_NB = _N // _B + _E
_NP = _NB * _B
_NW = 32
_TPW = _T // _NW
_CH = 32
_TCH = 8


def _route_body(idx_ref, cnt_ref, x_ref, meta_ref, pos_ref, xb_ref):
    i32, f32 = jnp.int32, jnp.float32
    xb_ref[...] = x_ref[...].astype(jnp.bfloat16)
    e_arr = idx_ref[...]
    cnt = cnt_ref[...]
    nbv = (cnt + (_B - 1)) >> _BSH
    pv_f = (nbv << _BSH).astype(f32)
    nbv_f = nbv.astype(f32)

    offs = []
    acc = jnp.zeros((1, 1), f32)
    for e in range(_E):
        offs.append(acc)
        acc = acc + pv_f[:, e:e + 1]
    cums = []
    cacc = jnp.zeros((1, 1), f32)
    for e in range(_E):
        cacc = cacc + nbv_f[:, e:e + 1]
        cums.append(cacc)

    U = (lax.broadcasted_iota(i32, (128, 128), 0)
         < lax.broadcasted_iota(i32, (128, 128), 1)).astype(f32)
    A = (lax.broadcasted_iota(i32, (32, 32), 1)
         < lax.broadcasted_iota(i32, (32, 32), 0)).astype(f32)
    pos_f = jnp.zeros((32, 128), f32)
    for e in range(_E):
        m = (e_arr == e).astype(f32)
        rank_row = jnp.dot(m, U, preferred_element_type=f32)
        rtot = jnp.sum(m, axis=1, keepdims=True)
        roff = jnp.dot(A, rtot, preferred_element_type=f32)
        pos_f = pos_f + m * (rank_row + roff + offs[e])
    pos_ref[...] = pos_f.astype(i32)

    b_iota = lax.broadcasted_iota(i32, (1, 128), 1)
    be = jnp.zeros((1, 128), i32)
    for e in range(_E):
        be = be + (b_iota >= cums[e].astype(i32)).astype(i32)
    be = jnp.minimum(be, _E - 1)
    nbu = cums[_E - 1].astype(i32)
    meta_ref[...] = jnp.where(b_iota < _NB, be, nbu)


_route = pl.pallas_call(
    _route_body,
    out_shape=(
        jax.ShapeDtypeStruct((1, 128), jnp.int32),
        jax.ShapeDtypeStruct((32, 128), jnp.int32),
        jax.ShapeDtypeStruct((_T, _D), jnp.bfloat16),
    ),
)


def _ycast_body(yb_ref, y_ref):
    y_ref[...] = yb_ref[...].astype(jnp.float32)


_ycast = pl.pallas_call(
    _ycast_body,
    grid=(8,),
    in_specs=[pl.BlockSpec((_T // 8, _D), lambda b: (b, 0))],
    out_specs=pl.BlockSpec((_T // 8, _D), lambda b: (b, 0)),
    out_shape=jax.ShapeDtypeStruct((_T, _D), jnp.float32),
)


def _gmm_body(be_ref, nbu_ref, xs_ref, wg_ref, wu_ref, wd_ref, ys_ref):
    del be_ref
    b = pl.program_id(0)

    @pl.when(b < nbu_ref[0])
    def _():
        bf16, f32 = jnp.bfloat16, jnp.float32
        xb = xs_ref[...]
        g = jnp.dot(xb, wg_ref[0].astype(bf16), preferred_element_type=f32)
        u = jnp.dot(xb, wu_ref[0].astype(bf16), preferred_element_type=f32)
        hb = (g / (1.0 + jnp.exp(-g)) * u).astype(bf16)
        ys_ref[...] = jnp.dot(hb, wd_ref[0].astype(bf16),
                              preferred_element_type=f32).astype(bf16)


_gmm = pl.pallas_call(
    _gmm_body,
    grid_spec=pltpu.PrefetchScalarGridSpec(
        num_scalar_prefetch=2,
        grid=(_NB,),
        in_specs=[
            pl.BlockSpec((_B, _D), lambda b, be, nbu: (b, 0)),
            pl.BlockSpec((1, _D, _H), lambda b, be, nbu: (be[b], 0, 0)),
            pl.BlockSpec((1, _D, _H), lambda b, be, nbu: (be[b], 0, 0)),
            pl.BlockSpec((1, _H, _D), lambda b, be, nbu: (be[b], 0, 0)),
        ],
        out_specs=pl.BlockSpec((_B, _D), lambda b, be, nbu: (b, 0)),
    ),
    out_shape=jax.ShapeDtypeStruct((_NP, _D), jnp.bfloat16),
    compiler_params=pltpu.CompilerParams(
        dimension_semantics=("arbitrary",),
        vmem_limit_bytes=120 * 1024 * 1024,
    ),
)


def _sc_mesh():
    return plsc.VectorSubcoreMesh(core_axis_name="c", subcore_axis_name="s")


def _dispatch_body(x_hbm, pe_hbm, po_hbm, xs_hbm, rows_v, pe_v, po_v, sem):
    wid = lax.axis_index("s") * 2 + lax.axis_index("c")
    tbase = wid * _TPW
    for c in range(_TPW // _CH):
        tb = tbase + c * _CH
        pltpu.sync_copy(x_hbm.at[pl.ds(tb, _CH)], rows_v)
        pltpu.sync_copy(pe_hbm.at[pl.ds(tb, _CH)], pe_v)
        pltpu.sync_copy(po_hbm.at[pl.ds(tb, _CH)], po_v)
        cp1 = pltpu.async_copy(rows_v, xs_hbm.at[pe_v], sem)
        cp2 = pltpu.async_copy(rows_v, xs_hbm.at[po_v], sem)
        cp1.wait()
        cp2.wait()


def _combine_body(ys_hbm, pos_hbm, w_hbm, y_hbm, pidx_a, pidx_b, w_v,
                  rows_a, rows_b, out_v, sem_a, sem_b):
    wid = lax.axis_index("s") * 2 + lax.axis_index("c")
    tbase = wid * _TPW
    nch = _TPW // _TCH
    pidx = (pidx_a, pidx_b)
    rows = (rows_a, rows_b)
    sems = (sem_a, sem_b)

    def start(c):
        i = c % 2
        tb = tbase + c * _TCH
        pltpu.sync_copy(pos_hbm.at[pl.ds(2 * tb, 2 * _TCH)], pidx[i])
        return pltpu.async_copy(ys_hbm.at[pidx[i]], rows[i], sems[i])

    cps = [start(0), None]
    for c in range(nch):
        i = c % 2
        tb = tbase + c * _TCH
        if c + 1 < nch:
            cps[(c + 1) % 2] = start(c + 1)
        pltpu.sync_copy(w_hbm.at[pl.ds(2 * tb, 2 * _TCH)], w_v)
        cps[i].wait()
        r = rows[i]
        wv = w_v[...]
        for tt in range(_TCH):
            w0 = jnp.full((16,), wv[2 * tt], jnp.float32)
            w1 = jnp.full((16,), wv[2 * tt + 1], jnp.float32)

            def body(s, carry, tt=tt, w0=w0, w1=w1, r=r):
                f32, i32 = jnp.float32, jnp.int32
                msk = jnp.int32(-65536)
                for u in range(8):
                    sl = pl.ds(u * 16, 16)
                    a = r[2 * tt, s, sl]
                    b = r[2 * tt + 1, s, sl]
                    alo = lax.bitcast_convert_type(a << 16, f32)
                    ahi = lax.bitcast_convert_type(a & msk, f32)
                    blo = lax.bitcast_convert_type(b << 16, f32)
                    bhi = lax.bitcast_convert_type(b & msk, f32)
                    re = w0 * alo + w1 * blo
                    ro = w0 * ahi + w1 * bhi
                    rei = lax.shift_right_logical(lax.bitcast_convert_type(re, i32), 16)
                    roi = lax.bitcast_convert_type(ro, i32) & msk
                    out_v[tt, s, sl] = roi | rei
                return carry

            lax.fori_loop(0, 8, body, 0)
        pltpu.sync_copy(out_v, y_hbm.at[pl.ds(tb, _TCH)])


@functools.cache
def _sc_kernels():
    mesh = _sc_mesh()
    dispatch = pl.kernel(
        _dispatch_body,
        out_type=jax.ShapeDtypeStruct((_NP, 8, 128), jnp.int32),
        mesh=mesh,
        scratch_types=[
            pltpu.VMEM((_CH, 8, 128), jnp.int32),
            pltpu.VMEM((_CH,), jnp.int32),
            pltpu.VMEM((_CH,), jnp.int32),
            pltpu.SemaphoreType.DMA,
        ],
    )
    combine = pl.kernel(
        _combine_body,
        out_type=jax.ShapeDtypeStruct((_T, 8, 128), jnp.int32),
        mesh=mesh,
        scratch_types=[
            pltpu.VMEM((2 * _TCH,), jnp.int32),
            pltpu.VMEM((2 * _TCH,), jnp.int32),
            pltpu.VMEM((2 * _TCH,), jnp.float32),
            pltpu.VMEM((2 * _TCH, 8, 128), jnp.int32),
            pltpu.VMEM((2 * _TCH, 8, 128), jnp.int32),
            pltpu.VMEM((_TCH, 8, 128), jnp.int32),
            pltpu.SemaphoreType.DMA,
            pltpu.SemaphoreType.DMA,
        ],
    )
    return dispatch, combine


def kernel(x, weights, indices, counts, W_gate, W_up, W_down):
    _dispatch, _combine = _sc_kernels()
    idx2d = indices.astype(jnp.int32).reshape(32, 128)
    cnt2d = counts.astype(jnp.int32).reshape(1, _E)
    meta, pos2d, xb = _route(idx2d, cnt2d, x)
    pos = pos2d.reshape(_N)
    be = meta[0, :_NB]
    nbu = meta[0, _NB:_NB + 1]
    posTK = pos2d.reshape(_T, _K)
    x32 = lax.bitcast_convert_type(
        xb.reshape(_T, 8, 128, 2), jnp.int32)
    xs32 = _dispatch(x32, posTK[:, 0], posTK[:, 1])
    xs = lax.bitcast_convert_type(
        xs32, jnp.bfloat16).reshape(_NP, _D)
    ys = _gmm(be, nbu, xs, W_gate, W_up, W_down)
    ys32 = lax.bitcast_convert_type(
        ys.reshape(_NP, 8, 128, 2), jnp.int32)
    y32 = _combine(ys32, pos, weights.reshape(_N))
    yb = lax.bitcast_convert_type(y32, jnp.bfloat16).reshape(_T, _D)
    return _ycast(yb)


# speedup vs baseline: 4.9060x; 4.9060x over previous
import functools

import jax
import jax.numpy as jnp
from jax import lax
from jax.experimental import pallas as pl
from jax.experimental.pallas import tpu as pltpu
from jax.experimental.pallas import tpu_sc as plsc

_T, _D, _H, _E, _K = 2048, 2048, 1024, 8, 2
_N = _T * _K
_B = 256
_BSH = _B.bit_length() - 1
_NB = _N // _B + _E
_NP = _NB * _B
_NW = 32
_TPW = _T // _NW
_CH = 32
_TCH = 8


def _route_body(idx_ref, cnt_ref, meta_ref, pos_ref):
    i32, f32 = jnp.int32, jnp.float32
    e_arr = idx_ref[...]
    cnt = cnt_ref[...]
    nbv = (cnt + (_B - 1)) >> _BSH
    pv_f = (nbv << _BSH).astype(f32)
    nbv_f = nbv.astype(f32)

    offs = []
    acc = jnp.zeros((1, 1), f32)
    for e in range(_E):
        offs.append(acc)
        acc = acc + pv_f[:, e:e + 1]
    cums = []
    cacc = jnp.zeros((1, 1), f32)
    for e in range(_E):
        cacc = cacc + nbv_f[:, e:e + 1]
        cums.append(cacc)

    U = (lax.broadcasted_iota(i32, (128, 128), 0)
         < lax.broadcasted_iota(i32, (128, 128), 1)).astype(f32)
    A = (lax.broadcasted_iota(i32, (32, 32), 1)
         < lax.broadcasted_iota(i32, (32, 32), 0)).astype(f32)
    pos_f = jnp.zeros((32, 128), f32)
    for e in range(_E):
        m = (e_arr == e).astype(f32)
        rank_row = jnp.dot(m, U, preferred_element_type=f32)
        rtot = jnp.sum(m, axis=1, keepdims=True)
        roff = jnp.dot(A, rtot, preferred_element_type=f32)
        pos_f = pos_f + m * (rank_row + roff + offs[e])
    pos_ref[...] = pos_f.astype(i32)

    b_iota = lax.broadcasted_iota(i32, (1, 128), 1)
    be = jnp.zeros((1, 128), i32)
    frun = jnp.zeros((1, 128), i32)
    nruns = jnp.zeros((1, 1), i32)
    for e in range(_E):
        cum_i = cums[e].astype(i32)
        nb_e = nbv[:, e:e + 1]
        be = be + (b_iota >= cum_i).astype(i32)
        frun = frun + ((b_iota == cum_i - nb_e) & (nb_e > 0)).astype(i32)
        nruns = nruns + (nb_e > 0).astype(i32)
    be = jnp.minimum(be, _E - 1)
    frun = jnp.minimum(frun, 1)
    nbu = cums[_E - 1].astype(i32)

    UI = (lax.broadcasted_iota(i32, (128, 128), 0)
          <= lax.broadcasted_iota(i32, (128, 128), 1)).astype(f32)
    rr = jnp.dot(frun.astype(f32), UI,
                 preferred_element_type=f32).astype(i32) - 1
    slot = rr & 1
    nxt_after = jnp.full((1, 1), _E - 1, i32)
    nxt = [None] * _E
    for e in range(_E - 1, -1, -1):
        nxt[e] = nxt_after
        nxt_after = jnp.where(nbv[:, e:e + 1] > 0,
                              jnp.full((1, 1), e, i32), nxt_after)
    nexp = jnp.zeros((1, 128), i32)
    for e in range(_E):
        nexp = nexp + (be == e).astype(i32) * nxt[e]
    ft = frun * ((rr + 1) < nruns).astype(i32)

    meta_ref[0:1, :] = jnp.where(b_iota < _NB, be, nbu)
    meta_ref[1:2, :] = frun
    meta_ref[2:3, :] = ft
    meta_ref[3:4, :] = slot
    meta_ref[4:5, :] = nexp
    meta_ref[5:6, :] = jnp.broadcast_to(nbu, (1, 128))


_route = pl.pallas_call(
    _route_body,
    out_shape=(
        jax.ShapeDtypeStruct((6, 128), jnp.int32),
        jax.ShapeDtypeStruct((32, 128), jnp.int32),
    ),
)


def _gmm_body(be_ref, wt_ref, ft_ref, sl_ref, nx_ref, nbu_ref,
              xs_ref, wg_hbm, wu_hbm, wd_hbm, ys_ref,
              wg_v, wu_v, wd_v, sem0, sem1):
    b = pl.program_id(0)
    bf16, f32 = jnp.bfloat16, jnp.float32
    sems = (sem0, sem1)

    def fetch(e, s):
        pltpu.make_async_copy(wg_hbm.at[e], wg_v.at[s], sems[s]).start()
        pltpu.make_async_copy(wu_hbm.at[e], wu_v.at[s], sems[s]).start()
        pltpu.make_async_copy(wd_hbm.at[e], wd_v.at[s], sems[s]).start()

    def wait(s):
        pltpu.make_async_copy(wg_hbm.at[0], wg_v.at[s], sems[s]).wait()
        pltpu.make_async_copy(wu_hbm.at[0], wu_v.at[s], sems[s]).wait()
        pltpu.make_async_copy(wd_hbm.at[0], wd_v.at[s], sems[s]).wait()

    first = wt_ref[b] == 1
    do_fetch = ft_ref[b] == 1
    s = sl_ref[b]

    @pl.when(b == 0)
    def _():
        fetch(be_ref[0], 0)

        @pl.when(ft_ref[0] == 1)
        def _():
            fetch(nx_ref[0], 1)

    @pl.when(first & (b > 0) & do_fetch & (s == 0))
    def _():
        fetch(nx_ref[b], 1)

    @pl.when(first & (b > 0) & do_fetch & (s == 1))
    def _():
        fetch(nx_ref[b], 0)

    @pl.when(first & (s == 0))
    def _():
        wait(0)

    @pl.when(first & (s == 1))
    def _():
        wait(1)

    @pl.when(b < nbu_ref[0])
    def _():
        xb = xs_ref[...].astype(bf16)
        wg = wg_v[s].astype(bf16)
        wu = wu_v[s].astype(bf16)
        g = jnp.dot(xb, wg, preferred_element_type=f32)
        u = jnp.dot(xb, wu, preferred_element_type=f32)
        hb = (g / (1.0 + jnp.exp(-g)) * u).astype(bf16)
        ys_ref[...] = jnp.dot(hb, wd_v[s].astype(bf16),
                              preferred_element_type=f32)


_gmm = pl.pallas_call(
    _gmm_body,
    grid_spec=pltpu.PrefetchScalarGridSpec(
        num_scalar_prefetch=6,
        grid=(_NB,),
        in_specs=[
            pl.BlockSpec(
                (_B, _D),
                lambda b, be, wt, ft, sl, nx, nbu:
                    (jnp.where(b < nbu[0], b, 0), 0)),
            pl.BlockSpec(memory_space=pltpu.MemorySpace.HBM),
            pl.BlockSpec(memory_space=pltpu.MemorySpace.HBM),
            pl.BlockSpec(memory_space=pltpu.MemorySpace.HBM),
        ],
        out_specs=pl.BlockSpec((_B, _D),
                               lambda b, be, wt, ft, sl, nx, nbu: (b, 0)),
        scratch_shapes=[
            pltpu.VMEM((2, _D, _H), jnp.float32),
            pltpu.VMEM((2, _D, _H), jnp.float32),
            pltpu.VMEM((2, _H, _D), jnp.float32),
            pltpu.SemaphoreType.DMA,
            pltpu.SemaphoreType.DMA,
        ],
    ),
    out_shape=jax.ShapeDtypeStruct((_NP, _D), jnp.float32),
    compiler_params=pltpu.CompilerParams(
        dimension_semantics=("arbitrary",),
        vmem_limit_bytes=120 * 1024 * 1024,
    ),
)


def _sc_mesh():
    return plsc.VectorSubcoreMesh(core_axis_name="c", subcore_axis_name="s")


def _dispatch_body(x_hbm, pe_hbm, po_hbm, xs_hbm, rows_v, pe_v, po_v, sem):
    wid = lax.axis_index("s") * 2 + lax.axis_index("c")
    tbase = wid * _TPW
    for c in range(_TPW // _CH):
        tb = tbase + c * _CH
        pltpu.sync_copy(x_hbm.at[pl.ds(tb, _CH)], rows_v)
        pltpu.sync_copy(pe_hbm.at[pl.ds(tb, _CH)], pe_v)
        pltpu.sync_copy(po_hbm.at[pl.ds(tb, _CH)], po_v)
        cp1 = pltpu.async_copy(rows_v, xs_hbm.at[pe_v], sem)
        cp2 = pltpu.async_copy(rows_v, xs_hbm.at[po_v], sem)
        cp1.wait()
        cp2.wait()


def _combine_body(ys_hbm, pos_hbm, w_hbm, y_hbm, pidx_a, pidx_b, w_v,
                  rows_a, rows_b, out_v, sem_a, sem_b):
    wid = lax.axis_index("s") * 2 + lax.axis_index("c")
    tbase = wid * _TPW
    nch = _TPW // _TCH
    pidx = (pidx_a, pidx_b)
    rows = (rows_a, rows_b)
    sems = (sem_a, sem_b)

    def start(c):
        i = c % 2
        tb = tbase + c * _TCH
        pltpu.sync_copy(pos_hbm.at[pl.ds(2 * tb, 2 * _TCH)], pidx[i])
        return pltpu.async_copy(ys_hbm.at[pidx[i]], rows[i], sems[i])

    cps = [start(0), None]
    for c in range(nch):
        i = c % 2
        tb = tbase + c * _TCH
        if c + 1 < nch:
            cps[(c + 1) % 2] = start(c + 1)
        pltpu.sync_copy(w_hbm.at[pl.ds(2 * tb, 2 * _TCH)], w_v)
        cps[i].wait()
        r = rows[i]
        wv = w_v[...]
        for tt in range(_TCH):
            w0 = jnp.full((16,), wv[2 * tt], jnp.float32)
            w1 = jnp.full((16,), wv[2 * tt + 1], jnp.float32)

            def body(dd, carry, tt=tt, w0=w0, w1=w1, r=r):
                base = dd * 128
                for u in range(8):
                    sl = pl.ds(base + u * 16, 16)
                    out_v[tt, sl] = (w0 * r[2 * tt, sl]
                                     + w1 * r[2 * tt + 1, sl])
                return carry

            lax.fori_loop(0, _D // 128, body, 0)
        pltpu.sync_copy(out_v, y_hbm.at[pl.ds(tb, _TCH)])


@functools.cache
def _sc_kernels():
    mesh = _sc_mesh()
    dispatch = pl.kernel(
        _dispatch_body,
        out_type=jax.ShapeDtypeStruct((_NP, _D), jnp.float32),
        mesh=mesh,
        scratch_types=[
            pltpu.VMEM((_CH, _D), jnp.float32),
            pltpu.VMEM((_CH,), jnp.int32),
            pltpu.VMEM((_CH,), jnp.int32),
            pltpu.SemaphoreType.DMA,
        ],
    )
    combine = pl.kernel(
        _combine_body,
        out_type=jax.ShapeDtypeStruct((_T, _D), jnp.float32),
        mesh=mesh,
        scratch_types=[
            pltpu.VMEM((2 * _TCH,), jnp.int32),
            pltpu.VMEM((2 * _TCH,), jnp.int32),
            pltpu.VMEM((2 * _TCH,), jnp.float32),
            pltpu.VMEM((2 * _TCH, _D), jnp.float32),
            pltpu.VMEM((2 * _TCH, _D), jnp.float32),
            pltpu.VMEM((_TCH, _D), jnp.float32),
            pltpu.SemaphoreType.DMA,
            pltpu.SemaphoreType.DMA,
        ],
    )
    return dispatch, combine


def kernel(x, weights, indices, counts, W_gate, W_up, W_down):
    _dispatch, _combine = _sc_kernels()
    idx2d = indices.astype(jnp.int32).reshape(32, 128)
    cnt2d = counts.astype(jnp.int32).reshape(1, _E)
    meta, pos2d = _route(idx2d, cnt2d)
    pos = pos2d.reshape(_N)
    be = meta[0, :_NB]
    wt = meta[1, :_NB]
    ft = meta[2, :_NB]
    sl = meta[3, :_NB]
    nx = meta[4, :_NB]
    nbu = meta[5, :1]
    posTK = pos2d.reshape(_T, _K)
    xs = _dispatch(x, posTK[:, 0], posTK[:, 1])
    ys = _gmm(be, wt, ft, sl, nx, nbu, xs, W_gate, W_up, W_down)
    return _combine(ys, pos, weights.reshape(_N))


# speedup vs baseline: 4.9606x; 1.0111x over previous
import functools

import jax
import jax.numpy as jnp
from jax import lax
from jax.experimental import pallas as pl
from jax.experimental.pallas import tpu as pltpu
from jax.experimental.pallas import tpu_sc as plsc

_T, _D, _H, _E, _K = 2048, 2048, 1024, 8, 2
_N = _T * _K
_B = 256
_BSH = _B.bit_length() - 1
_NB = _N // _B + _E
_NP = _NB * _B
_NW = 32
_TPW = _T // _NW
_CH = 16
_TCH = 8


def _route_body(idx_ref, cnt_ref, meta_ref, pos_ref):
    i32, f32 = jnp.int32, jnp.float32
    e_arr = idx_ref[...]
    cnt = cnt_ref[...]
    nbv = (cnt + (_B - 1)) >> _BSH
    pv_f = (nbv << _BSH).astype(f32)
    nbv_f = nbv.astype(f32)

    offs = []
    acc = jnp.zeros((1, 1), f32)
    for e in range(_E):
        offs.append(acc)
        acc = acc + pv_f[:, e:e + 1]
    cums = []
    cacc = jnp.zeros((1, 1), f32)
    for e in range(_E):
        cacc = cacc + nbv_f[:, e:e + 1]
        cums.append(cacc)

    U = (lax.broadcasted_iota(i32, (128, 128), 0)
         < lax.broadcasted_iota(i32, (128, 128), 1)).astype(f32)
    A = (lax.broadcasted_iota(i32, (32, 32), 1)
         < lax.broadcasted_iota(i32, (32, 32), 0)).astype(f32)
    pos_f = jnp.zeros((32, 128), f32)
    for e in range(_E):
        m = (e_arr == e).astype(f32)
        rank_row = jnp.dot(m, U, preferred_element_type=f32)
        rtot = jnp.sum(m, axis=1, keepdims=True)
        roff = jnp.dot(A, rtot, preferred_element_type=f32)
        pos_f = pos_f + m * (rank_row + roff + offs[e])
    pos_ref[...] = pos_f.astype(i32)

    b_iota = lax.broadcasted_iota(i32, (1, 128), 1)
    be = jnp.zeros((1, 128), i32)
    frun = jnp.zeros((1, 128), i32)
    nruns = jnp.zeros((1, 1), i32)
    for e in range(_E):
        cum_i = cums[e].astype(i32)
        nb_e = nbv[:, e:e + 1]
        be = be + (b_iota >= cum_i).astype(i32)
        frun = frun + ((b_iota == cum_i - nb_e) & (nb_e > 0)).astype(i32)
        nruns = nruns + (nb_e > 0).astype(i32)
    be = jnp.minimum(be, _E - 1)
    frun = jnp.minimum(frun, 1)
    nbu = cums[_E - 1].astype(i32)

    UI = (lax.broadcasted_iota(i32, (128, 128), 0)
          <= lax.broadcasted_iota(i32, (128, 128), 1)).astype(f32)
    rr = jnp.dot(frun.astype(f32), UI,
                 preferred_element_type=f32).astype(i32) - 1
    slot = rr & 1
    nxt_after = jnp.full((1, 1), _E - 1, i32)
    nxt = [None] * _E
    for e in range(_E - 1, -1, -1):
        nxt[e] = nxt_after
        nxt_after = jnp.where(nbv[:, e:e + 1] > 0,
                              jnp.full((1, 1), e, i32), nxt_after)
    nexp = jnp.zeros((1, 128), i32)
    for e in range(_E):
        nexp = nexp + (be == e).astype(i32) * nxt[e]
    ft = frun * ((rr + 1) < nruns).astype(i32)

    meta_ref[0:1, :] = jnp.where(b_iota < _NB, be, nbu)
    meta_ref[1:2, :] = frun
    meta_ref[2:3, :] = ft
    meta_ref[3:4, :] = slot
    meta_ref[4:5, :] = nexp
    meta_ref[5:6, :] = jnp.broadcast_to(nbu, (1, 128))


_route = pl.pallas_call(
    _route_body,
    out_shape=(
        jax.ShapeDtypeStruct((6, 128), jnp.int32),
        jax.ShapeDtypeStruct((32, 128), jnp.int32),
    ),
)


def _gmm_body(be_ref, wt_ref, ft_ref, sl_ref, nx_ref, nbu_ref,
              xs_ref, wg_hbm, wu_hbm, wd_hbm, ys_ref,
              wg_v, wu_v, wd_v, sg0, sg1, su0, su1, sd0, sd1):
    b = pl.program_id(0)
    bf16, f32 = jnp.bfloat16, jnp.float32
    sg = (sg0, sg1)
    su = (su0, su1)
    sd = (sd0, sd1)

    def fetch(e, s):
        pltpu.make_async_copy(wg_hbm.at[e], wg_v.at[s], sg[s]).start()
        pltpu.make_async_copy(wu_hbm.at[e], wu_v.at[s], su[s]).start()
        pltpu.make_async_copy(wd_hbm.at[e], wd_v.at[s], sd[s]).start()

    first = wt_ref[b] == 1
    do_fetch = ft_ref[b] == 1
    s = sl_ref[b]

    @pl.when(b == 0)
    def _():
        fetch(be_ref[0], 0)

        @pl.when(ft_ref[0] == 1)
        def _():
            fetch(nx_ref[0], 1)

    @pl.when(first & (b > 0) & do_fetch & (s == 0))
    def _():
        fetch(nx_ref[b], 1)

    @pl.when(first & (b > 0) & do_fetch & (s == 1))
    def _():
        fetch(nx_ref[b], 0)

    for ss in (0, 1):
        @pl.when(first & (s == ss))
        def _(ss=ss):
            pltpu.make_async_copy(wg_hbm.at[0], wg_v.at[ss], sg[ss]).wait()
            pltpu.make_async_copy(wu_hbm.at[0], wu_v.at[ss], su[ss]).wait()

    @pl.when(b < nbu_ref[0])
    def _():
        xb = xs_ref[...].astype(bf16)
        g = jnp.dot(xb, wg_v[s].astype(bf16), preferred_element_type=f32)
        u = jnp.dot(xb, wu_v[s].astype(bf16), preferred_element_type=f32)
        hb = (g / (1.0 + jnp.exp(-g)) * u).astype(bf16)
        for ss in (0, 1):
            @pl.when(first & (s == ss))
            def _(ss=ss):
                pltpu.make_async_copy(wd_hbm.at[0], wd_v.at[ss],
                                      sd[ss]).wait()
        ys_ref[...] = jnp.dot(hb, wd_v[s].astype(bf16),
                              preferred_element_type=f32)


_gmm = pl.pallas_call(
    _gmm_body,
    grid_spec=pltpu.PrefetchScalarGridSpec(
        num_scalar_prefetch=6,
        grid=(_NB,),
        in_specs=[
            pl.BlockSpec(
                (_B, _D),
                lambda b, be, wt, ft, sl, nx, nbu:
                    (jnp.where(b < nbu[0], b, 0), 0)),
            pl.BlockSpec(memory_space=pltpu.MemorySpace.HBM),
            pl.BlockSpec(memory_space=pltpu.MemorySpace.HBM),
            pl.BlockSpec(memory_space=pltpu.MemorySpace.HBM),
        ],
        out_specs=pl.BlockSpec(
            (_B, _D),
            lambda b, be, wt, ft, sl, nx, nbu:
                (jnp.where(b < nbu[0], b, _NB - 1), 0)),
        scratch_shapes=[
            pltpu.VMEM((2, _D, _H), jnp.float32),
            pltpu.VMEM((2, _D, _H), jnp.float32),
            pltpu.VMEM((2, _H, _D), jnp.float32),
            pltpu.SemaphoreType.DMA,
            pltpu.SemaphoreType.DMA,
            pltpu.SemaphoreType.DMA,
            pltpu.SemaphoreType.DMA,
            pltpu.SemaphoreType.DMA,
            pltpu.SemaphoreType.DMA,
        ],
    ),
    out_shape=jax.ShapeDtypeStruct((_NP, _D), jnp.float32),
    compiler_params=pltpu.CompilerParams(
        dimension_semantics=("arbitrary",),
        vmem_limit_bytes=120 * 1024 * 1024,
    ),
)


def _sc_mesh():
    return plsc.VectorSubcoreMesh(core_axis_name="c", subcore_axis_name="s")


def _dispatch_body(x_hbm, pe_hbm, po_hbm, xs_hbm, rows_a, rows_b,
                   pe_v, po_v, sem_a, sem_b, sco_a, sco_b):
    wid = lax.axis_index("s") * 2 + lax.axis_index("c")
    tbase = wid * _TPW
    nch = _TPW // _CH
    rows = (rows_a, rows_b)
    sin = (sem_a, sem_b)
    sout = (sco_a, sco_b)

    def start_in(c):
        i = c % 2
        return pltpu.async_copy(x_hbm.at[pl.ds(tbase + c * _CH, _CH)],
                                rows[i], sin[i])

    pltpu.sync_copy(pe_hbm.at[pl.ds(wid * nch, nch)], pe_v)
    pltpu.sync_copy(po_hbm.at[pl.ds(wid * nch, nch)], po_v)
    cin = [start_in(0), None]
    couts = []
    for c in range(nch):
        i = c % 2
        if c + 1 < nch:
            cin[(c + 1) % 2] = start_in(c + 1)
        cin[i].wait()
        if c >= 2:
            for cp in couts[2 * (c - 2):2 * (c - 1)]:
                cp.wait()
        couts.append(pltpu.async_copy(rows[i], xs_hbm.at[pe_v.at[c]],
                                      sout[i]))
        couts.append(pltpu.async_copy(rows[i], xs_hbm.at[po_v.at[c]],
                                      sout[i]))
    for cp in couts[2 * (nch - 2):]:
        cp.wait()


def _combine_body(ys_hbm, pos_hbm, w_hbm, y_hbm, pidx_a, pidx_b, w_v,
                  rows_a, rows_b, out_v, sem_a, sem_b):
    wid = lax.axis_index("s") * 2 + lax.axis_index("c")
    tbase = wid * _TPW
    nch = _TPW // _TCH
    pidx = (pidx_a, pidx_b)
    rows = (rows_a, rows_b)
    sems = (sem_a, sem_b)

    def start(c):
        i = c % 2
        tb = tbase + c * _TCH
        pltpu.sync_copy(pos_hbm.at[pl.ds(2 * tb, 2 * _TCH)], pidx[i])
        return pltpu.async_copy(ys_hbm.at[pidx[i]], rows[i], sems[i])

    cps = [start(0), None]
    for c in range(nch):
        i = c % 2
        tb = tbase + c * _TCH
        if c + 1 < nch:
            cps[(c + 1) % 2] = start(c + 1)
        pltpu.sync_copy(w_hbm.at[pl.ds(2 * tb, 2 * _TCH)], w_v)
        cps[i].wait()
        r = rows[i]
        wv = w_v[...]
        for tt in range(_TCH):
            w0 = jnp.full((16,), wv[2 * tt], jnp.float32)
            w1 = jnp.full((16,), wv[2 * tt + 1], jnp.float32)

            def body(dd, carry, tt=tt, w0=w0, w1=w1, r=r):
                base = dd * 128
                for u in range(8):
                    sl = pl.ds(base + u * 16, 16)
                    out_v[tt, sl] = (w0 * r[2 * tt, sl]
                                     + w1 * r[2 * tt + 1, sl])
                return carry

            lax.fori_loop(0, _D // 128, body, 0)
        pltpu.sync_copy(out_v, y_hbm.at[pl.ds(tb, _TCH)])


@functools.cache
def _sc_kernels():
    mesh = _sc_mesh()
    dispatch = pl.kernel(
        _dispatch_body,
        out_type=jax.ShapeDtypeStruct((_NP, _D), jnp.float32),
        mesh=mesh,
        scratch_types=[
            pltpu.VMEM((_CH, _D), jnp.float32),
            pltpu.VMEM((_CH, _D), jnp.float32),
            pltpu.VMEM((_TPW // _CH, _CH), jnp.int32),
            pltpu.VMEM((_TPW // _CH, _CH), jnp.int32),
            pltpu.SemaphoreType.DMA,
            pltpu.SemaphoreType.DMA,
            pltpu.SemaphoreType.DMA,
            pltpu.SemaphoreType.DMA,
        ],
    )
    combine = pl.kernel(
        _combine_body,
        out_type=jax.ShapeDtypeStruct((_T, _D), jnp.float32),
        mesh=mesh,
        scratch_types=[
            pltpu.VMEM((2 * _TCH,), jnp.int32),
            pltpu.VMEM((2 * _TCH,), jnp.int32),
            pltpu.VMEM((2 * _TCH,), jnp.float32),
            pltpu.VMEM((2 * _TCH, _D), jnp.float32),
            pltpu.VMEM((2 * _TCH, _D), jnp.float32),
            pltpu.VMEM((_TCH, _D), jnp.float32),
            pltpu.SemaphoreType.DMA,
            pltpu.SemaphoreType.DMA,
        ],
    )
    return dispatch, combine


def kernel(x, weights, indices, counts, W_gate, W_up, W_down):
    _dispatch, _combine = _sc_kernels()
    idx2d = indices.astype(jnp.int32).reshape(32, 128)
    cnt2d = counts.astype(jnp.int32).reshape(1, _E)
    meta, pos2d = _route(idx2d, cnt2d)
    pos = pos2d.reshape(_N)
    be = meta[0, :_NB]
    wt = meta[1, :_NB]
    ft = meta[2, :_NB]
    sl = meta[3, :_NB]
    nx = meta[4, :_NB]
    nbu = meta[5, :1]
    posTK = pos2d.reshape(_T, _K)
    pe2 = posTK[:, 0].reshape(_T // _CH, _CH)
    po2 = posTK[:, 1].reshape(_T // _CH, _CH)
    xs = _dispatch(x, pe2, po2)
    ys = _gmm(be, wt, ft, sl, nx, nbu, xs, W_gate, W_up, W_down)
    return _combine(ys, pos, weights.reshape(_N))
